# baseline (speedup 1.0000x reference)
import functools
import math

import jax
import jax.numpy as jnp
from jax import lax
from jax.experimental import pallas as pl
from jax.experimental.pallas import tpu as pltpu
from jax.experimental.pallas import tpu_sc as plsc

HIDDEN = 1024
MEM = 512
BUCKETS = 100000
SLOTS = 8
SLOT_DIM = MEM // SLOTS
N = 16384

NC = 2
NS = 16
NW = NC * NS
ROWS_PER_W = N // NW
CHUNK = 128
CHUNKS_PER_SLOT = ROWS_PER_W // CHUNK


def _sc_gather(table_hbm, idx_hbm, out_hbm, idx_v, rows_v, sem):
    wid = lax.axis_index("s") * NC + lax.axis_index("c")
    n0 = wid * ROWS_PER_W
    pltpu.sync_copy(idx_hbm.at[:, pl.ds(n0, ROWS_PER_W)], idx_v)

    def body(j, carry):
        s = j // CHUNKS_PER_SLOT
        k = j % CHUNKS_PER_SLOT
        pltpu.async_copy(
            table_hbm.at[s].at[idx_v.at[s, pl.ds(k * CHUNK, CHUNK)]],
            rows_v, sem).wait()
        pltpu.sync_copy(
            rows_v, out_hbm.at[s, pl.ds(n0 + k * CHUNK, CHUNK), :])
        return carry

    lax.fori_loop(0, SLOTS * CHUNKS_PER_SLOT, body, 0)


def _make_gather_call():
    return functools.partial(
        pl.kernel,
        out_type=jax.ShapeDtypeStruct((SLOTS, N, 2 * SLOT_DIM), jnp.float32),
        mesh=plsc.VectorSubcoreMesh(core_axis_name="c", subcore_axis_name="s",
                                    num_cores=NC, num_subcores=NS),
        scratch_types=[
            pltpu.VMEM((SLOTS, ROWS_PER_W), jnp.int32),
            pltpu.VMEM((CHUNK, 2 * SLOT_DIM), jnp.float32),
            pltpu.SemaphoreType.DMA,
        ],
        compiler_params=pltpu.CompilerParams(use_tc_tiling_on_sc=True),
    )(_sc_gather)


def _pack_body(in_ref, out_ref):
    x = in_ref[0]
    t = x.T
    out_ref[0] = jnp.concatenate([t, jnp.zeros_like(t)], axis=1)


def _dense_body(hid_ref, mem_ref, wkt_ref, wvt_ref, qn_ref, kn_ref,
                vn_ref, cw_ref, cb_ref, out_ref):
    eps = 1e-8
    q = hid_ref[...]
    q = q * lax.rsqrt(jnp.mean(q * q, axis=-1, keepdims=True) + eps)
    q = q * qn_ref[...]
    x = mem_ref[...]
    m = jnp.concatenate([x[s, :, :SLOT_DIM] for s in range(SLOTS)], axis=-1)
    m_hi = m.astype(jnp.bfloat16)

    def matmul(w):
        return jnp.dot(m_hi, w.astype(jnp.bfloat16),
                       preferred_element_type=jnp.float32)

    k = matmul(wkt_ref[...])
    k = k * lax.rsqrt(jnp.mean(k * k, axis=-1, keepdims=True) + eps)
    k = k * kn_ref[...]
    v = matmul(wvt_ref[...])
    v = v * lax.rsqrt(jnp.mean(v * v, axis=-1, keepdims=True) + eps)
    v = v * vn_ref[...]
    logits = jnp.sum(q * k, axis=-1, keepdims=True) * (1.0 / math.sqrt(HIDDEN))
    alpha = jax.nn.sigmoid(logits)
    g = alpha * v
    co = g * cw_ref[...] + cb_ref[...]
    out_ref[...] = co * jax.nn.sigmoid(co) + g


def kernel(hidden, batch_ngram_bucket_ids, tables, Wk, Wv, qn_w, kn_w, vn_w,
           conv_w, conv_b):
    idx = jnp.asarray(batch_ngram_bucket_ids, jnp.int32).T

    cb = 1280
    tabT = jnp.transpose(tables, (0, 2, 1))
    packed = pl.pallas_call(
        _pack_body,
        grid=(SLOTS, -(-BUCKETS // cb)),
        in_specs=[pl.BlockSpec((1, SLOT_DIM, cb), lambda s, j: (s, 0, j))],
        out_specs=pl.BlockSpec((1, cb, 2 * SLOT_DIM), lambda s, j: (s, j, 0)),
        out_shape=jax.ShapeDtypeStruct((SLOTS, BUCKETS, 2 * SLOT_DIM),
                                       jnp.float32),
    )(tabT)

    rows = _make_gather_call()(packed, idx)

    bn = 1024
    grid = (N // bn,)
    full = lambda i: (0, 0)
    vec = lambda x: x.reshape(1, HIDDEN)
    out = pl.pallas_call(
        _dense_body,
        grid=grid,
        in_specs=[
            pl.BlockSpec((bn, HIDDEN), lambda i: (i, 0)),
            pl.BlockSpec((SLOTS, bn, 2 * SLOT_DIM), lambda i: (0, i, 0)),
            pl.BlockSpec((MEM, HIDDEN), full),
            pl.BlockSpec((MEM, HIDDEN), full),
            pl.BlockSpec((1, HIDDEN), full),
            pl.BlockSpec((1, HIDDEN), full),
            pl.BlockSpec((1, HIDDEN), full),
            pl.BlockSpec((1, HIDDEN), full),
            pl.BlockSpec((1, HIDDEN), full),
        ],
        out_specs=pl.BlockSpec((bn, HIDDEN), lambda i: (i, 0)),
        out_shape=jax.ShapeDtypeStruct((N, HIDDEN), jnp.float32),
    )(hidden, rows, Wk.T, Wv.T, vec(qn_w), vec(kn_w), vec(vn_w),
      vec(conv_w[:, 0, 2]), vec(conv_b))
    return out


# speedup vs baseline: 1.0132x; 1.0132x over previous
import functools
import math

import jax
import jax.numpy as jnp
from jax import lax
from jax.experimental import pallas as pl
from jax.experimental.pallas import tpu as pltpu
from jax.experimental.pallas import tpu_sc as plsc

HIDDEN = 1024
MEM = 512
BUCKETS = 100000
SLOTS = 8
SLOT_DIM = MEM // SLOTS
N = 16384

NC = 2
NS = 16
NW = NC * NS
ROWS_PER_W = N // NW
CHUNK = 128
CHUNKS_PER_SLOT = ROWS_PER_W // CHUNK


def _sc_gather(table_hbm, idx_hbm, out_hbm, idx_v, rows_v, sem):
    wid = lax.axis_index("s") * NC + lax.axis_index("c")
    n0 = wid * ROWS_PER_W
    pltpu.sync_copy(idx_hbm.at[:, pl.ds(n0, ROWS_PER_W)], idx_v)

    def body(j, carry):
        s = j // CHUNKS_PER_SLOT
        k = j % CHUNKS_PER_SLOT
        pltpu.async_copy(
            table_hbm.at[s].at[idx_v.at[s, pl.ds(k * CHUNK, CHUNK)]],
            rows_v, sem).wait()
        pltpu.sync_copy(
            rows_v, out_hbm.at[s, pl.ds(n0 + k * CHUNK, CHUNK), :])
        return carry

    lax.fori_loop(0, SLOTS * CHUNKS_PER_SLOT, body, 0)


def _make_gather_call():
    return functools.partial(
        pl.kernel,
        out_type=jax.ShapeDtypeStruct((SLOTS, N, 2 * SLOT_DIM), jnp.float32),
        mesh=plsc.VectorSubcoreMesh(core_axis_name="c", subcore_axis_name="s",
                                    num_cores=NC, num_subcores=NS),
        scratch_types=[
            pltpu.VMEM((SLOTS, ROWS_PER_W), jnp.int32),
            pltpu.VMEM((CHUNK, 2 * SLOT_DIM), jnp.float32),
            pltpu.SemaphoreType.DMA,
        ],
        compiler_params=pltpu.CompilerParams(use_tc_tiling_on_sc=True),
    )(_sc_gather)


def _dense_body(hid_ref, mem_ref, par_ref, wkt_ref, wvt_ref, qn_ref, kn_ref,
                vn_ref, cw_ref, cb_ref, out_ref):
    eps = 1e-8
    q = hid_ref[...]
    q = q * lax.rsqrt(jnp.mean(q * q, axis=-1, keepdims=True) + eps)
    q = q * qn_ref[...]
    x = mem_ref[...]
    p = par_ref[...]
    m = jnp.concatenate(
        [jnp.where(p[:, s:s + 1] > 0, x[s, :, SLOT_DIM:], x[s, :, :SLOT_DIM])
         for s in range(SLOTS)], axis=-1)
    m_hi = m.astype(jnp.bfloat16)

    def matmul(w):
        return jnp.dot(m_hi, w.astype(jnp.bfloat16),
                       preferred_element_type=jnp.float32)

    k = matmul(wkt_ref[...])
    k = k * lax.rsqrt(jnp.mean(k * k, axis=-1, keepdims=True) + eps)
    k = k * kn_ref[...]
    v = matmul(wvt_ref[...])
    v = v * lax.rsqrt(jnp.mean(v * v, axis=-1, keepdims=True) + eps)
    v = v * vn_ref[...]
    logits = jnp.sum(q * k, axis=-1, keepdims=True) * (1.0 / math.sqrt(HIDDEN))
    alpha = jax.nn.sigmoid(logits)
    g = alpha * v
    co = g * cw_ref[...] + cb_ref[...]
    out_ref[...] = co * jax.nn.sigmoid(co) + g


def kernel(hidden, batch_ngram_bucket_ids, tables, Wk, Wv, qn_w, kn_w, vn_w,
           conv_w, conv_b):
    ids32 = jnp.asarray(batch_ngram_bucket_ids, jnp.int32)
    idx = (ids32 >> 1).T
    par = (ids32 & 1).astype(jnp.float32)
    tab128 = tables.reshape(SLOTS, BUCKETS // 2, 2 * SLOT_DIM)

    rows = _make_gather_call()(tab128, idx)

    bn = 1024
    grid = (N // bn,)
    full = lambda i: (0, 0)
    vec = lambda x: x.reshape(1, HIDDEN)
    out = pl.pallas_call(
        _dense_body,
        grid=grid,
        in_specs=[
            pl.BlockSpec((bn, HIDDEN), lambda i: (i, 0)),
            pl.BlockSpec((SLOTS, bn, 2 * SLOT_DIM), lambda i: (0, i, 0)),
            pl.BlockSpec((bn, SLOTS), lambda i: (i, 0)),
            pl.BlockSpec((MEM, HIDDEN), full),
            pl.BlockSpec((MEM, HIDDEN), full),
            pl.BlockSpec((1, HIDDEN), full),
            pl.BlockSpec((1, HIDDEN), full),
            pl.BlockSpec((1, HIDDEN), full),
            pl.BlockSpec((1, HIDDEN), full),
            pl.BlockSpec((1, HIDDEN), full),
        ],
        out_specs=pl.BlockSpec((bn, HIDDEN), lambda i: (i, 0)),
        out_shape=jax.ShapeDtypeStruct((N, HIDDEN), jnp.float32),
    )(hidden, rows, par, Wk.T, Wv.T, vec(qn_w), vec(kn_w), vec(vn_w),
      vec(conv_w[:, 0, 2]), vec(conv_b))
    return out


# speedup vs baseline: 1.1546x; 1.1396x over previous
import functools
import math

import jax
import jax.numpy as jnp
from jax import lax
from jax.experimental import pallas as pl
from jax.experimental.pallas import tpu as pltpu
from jax.experimental.pallas import tpu_sc as plsc

HIDDEN = 1024
MEM = 512
BUCKETS = 100000
SLOTS = 8
SLOT_DIM = MEM // SLOTS
N = 16384

NC = 2
NS = 16
NW = NC * NS
ROWS_PER_W = N // NW
CHUNK = 128
CHUNKS_PER_SLOT = ROWS_PER_W // CHUNK


def _sc_gather(table_hbm, idx_hbm, out_hbm, idx_v, rows_v, sem):
    wid = lax.axis_index("s") * NC + lax.axis_index("c")
    n0 = wid * ROWS_PER_W
    pltpu.sync_copy(idx_hbm.at[:, pl.ds(n0, ROWS_PER_W)], idx_v)

    def body(j, carry):
        s = j // CHUNKS_PER_SLOT
        k = j % CHUNKS_PER_SLOT
        pltpu.async_copy(
            table_hbm.at[s].at[idx_v.at[s, pl.ds(k * CHUNK, CHUNK)]],
            rows_v, sem).wait()
        pltpu.sync_copy(
            rows_v, out_hbm.at[s, pl.ds(n0 + k * CHUNK, CHUNK), :])
        return carry

    lax.fori_loop(0, SLOTS * CHUNKS_PER_SLOT, body, 0)


def _make_gather_call():
    return functools.partial(
        pl.kernel,
        out_type=jax.ShapeDtypeStruct((SLOTS, N, 2 * SLOT_DIM), jnp.float32),
        mesh=plsc.VectorSubcoreMesh(core_axis_name="c", subcore_axis_name="s",
                                    num_cores=NC, num_subcores=NS),
        scratch_types=[
            pltpu.VMEM((SLOTS, ROWS_PER_W), jnp.int32),
            pltpu.VMEM((CHUNK, 2 * SLOT_DIM), jnp.float32),
            pltpu.SemaphoreType.DMA,
        ],
        compiler_params=pltpu.CompilerParams(use_tc_tiling_on_sc=True),
    )(_sc_gather)


def _dense_body(hid_ref, mem_ref, wkt_ref, wvt_ref, qn_ref, kn_ref,
                vn_ref, cw_ref, cb_ref, out_ref):
    eps = 1e-8
    q = hid_ref[...]
    q = q * lax.rsqrt(jnp.mean(q * q, axis=-1, keepdims=True) + eps)
    q = q * qn_ref[...]
    x = mem_ref[...]
    m = jnp.concatenate([x[s, :, :SLOT_DIM] for s in range(SLOTS)], axis=-1)
    m_hi = m.astype(jnp.bfloat16)

    def matmul(w):
        return jnp.dot(m_hi, w.astype(jnp.bfloat16),
                       preferred_element_type=jnp.float32)

    k = matmul(wkt_ref[...])
    k = k * lax.rsqrt(jnp.mean(k * k, axis=-1, keepdims=True) + eps)
    k = k * kn_ref[...]
    v = matmul(wvt_ref[...])
    v = v * lax.rsqrt(jnp.mean(v * v, axis=-1, keepdims=True) + eps)
    v = v * vn_ref[...]
    logits = jnp.sum(q * k, axis=-1, keepdims=True) * (1.0 / math.sqrt(HIDDEN))
    alpha = jax.nn.sigmoid(logits)
    g = alpha * v
    co = g * cw_ref[...] + cb_ref[...]
    out_ref[...] = co * jax.nn.sigmoid(co) + g


def kernel(hidden, batch_ngram_bucket_ids, tables, Wk, Wv, qn_w, kn_w, vn_w,
           conv_w, conv_b):
    idx = jnp.asarray(batch_ngram_bucket_ids, jnp.int32).T
    tab128 = jnp.pad(tables, ((0, 0), (0, 0), (0, SLOT_DIM)))

    rows = _make_gather_call()(tab128, idx)

    bn = 1024
    grid = (N // bn,)
    full = lambda i: (0, 0)
    vec = lambda x: x.reshape(1, HIDDEN)
    out = pl.pallas_call(
        _dense_body,
        grid=grid,
        in_specs=[
            pl.BlockSpec((bn, HIDDEN), lambda i: (i, 0)),
            pl.BlockSpec((SLOTS, bn, 2 * SLOT_DIM), lambda i: (0, i, 0)),
            pl.BlockSpec((MEM, HIDDEN), full),
            pl.BlockSpec((MEM, HIDDEN), full),
            pl.BlockSpec((1, HIDDEN), full),
            pl.BlockSpec((1, HIDDEN), full),
            pl.BlockSpec((1, HIDDEN), full),
            pl.BlockSpec((1, HIDDEN), full),
            pl.BlockSpec((1, HIDDEN), full),
        ],
        out_specs=pl.BlockSpec((bn, HIDDEN), lambda i: (i, 0)),
        out_shape=jax.ShapeDtypeStruct((N, HIDDEN), jnp.float32),
    )(hidden, rows, Wk.T, Wv.T, vec(qn_w), vec(kn_w), vec(vn_w),
      vec(conv_w[:, 0, 2]), vec(conv_b))
    return out
